# baseline (speedup 1.0000x reference)
import jax
import jax.numpy as jnp
from jax import lax
from jax.experimental import pallas as pl
from jax.experimental.pallas import tpu as pltpu
from jax.experimental.pallas import tpu_sc as plsc

B = 16384
F = 16
FH = F // 2
NC = 2
NS = 16
L = 16
NPAIR = NC * (NS // 2)
EPP = B // NPAIR
NWAVE = EPP // L
NGR = EPP // L


def _fm_body(user_hbm, item_hbm, uembT_hbm, iembT_hbm, biasT_hbm, w0_hbm,
             dummy_hbm, out_hbm,
             idx_u, idx_i, ub0, ub1, ib0, ib1, bb0, bb1,
             part_v, cb0, cb1, w0_v, out_v, sp_part, sem0, sem1):
    s = lax.axis_index("s")
    c = lax.axis_index("c")
    role = lax.shift_right_logical(s, 3)
    q = s & 7
    pair = c * 8 + q
    base = pair * EPP
    ro = pl.multiple_of(lax.shift_left(role, 3), 8)

    pltpu.sync_copy(user_hbm.at[pl.ds(base, EPP)], idx_u)
    pltpu.sync_copy(item_hbm.at[pl.ds(base, EPP)], idx_i)
    pltpu.sync_copy(w0_hbm, w0_v)

    w0vec = w0_v[...]
    mask127 = jnp.full((L,), 127, jnp.int32)
    ubuf = (ub0, ub1)
    ibuf = (ib0, ib1)
    bbuf = (bb0, bb1)
    sems = (sem0, sem1)

    def fire(w, p):
        uvec = idx_u[pl.ds(w * L, L)]
        ivec = idx_i[pl.ds(w * L, L)]
        bsel = jnp.where(role == 0, uvec, ivec)
        for j in range(L):
            ru = uvec[j]
            ri = ivec[j]
            rb = bsel[j]
            rbu = pl.multiple_of(
                lax.shift_left(lax.shift_right_logical(ru, 7), 7), 128)
            rbi = pl.multiple_of(
                lax.shift_left(lax.shift_right_logical(ri, 7), 7), 128)
            rbb = pl.multiple_of(
                lax.shift_left(lax.shift_right_logical(rb, 7), 7), 128)
            pltpu.async_copy(
                uembT_hbm.at[pl.ds(ro, FH), pl.ds(rbu, 128)], ubuf[p].at[j], sems[p])
            pltpu.async_copy(
                iembT_hbm.at[pl.ds(ro, FH), pl.ds(rbi, 128)], ibuf[p].at[j], sems[p])
            pltpu.async_copy(
                biasT_hbm.at[:, pl.ds(rbb, 128)], bbuf[p].at[pl.ds(j, 1)], sems[p])

    def drain(p):
        pltpu.make_async_copy(dummy_hbm, ubuf[p], sems[p]).wait()
        pltpu.make_async_copy(dummy_hbm, ibuf[p], sems[p]).wait()
        pltpu.make_async_copy(biasT_hbm.at[:, pl.ds(0, 128)],
                              bbuf[p].at[pl.ds(0, 1)], sems[p]).wait()
        for j in range(1, L):
            pltpu.make_async_copy(biasT_hbm.at[:, pl.ds(0, 128)],
                                  bbuf[p].at[pl.ds(j, 1)], sems[p]).wait()

    def compute(w, p):
        uvec = idx_u[pl.ds(w * L, L)]
        ivec = idx_i[pl.ds(w * L, L)]
        lanes_u = uvec & mask127
        lanes_i = ivec & mask127
        lanes_b = jnp.where(role == 0, lanes_u, lanes_i)
        el = lax.iota(jnp.int32, L)
        acc = jnp.zeros((L,), jnp.float32)
        for f in range(FH):
            fv = jnp.full((L,), f, jnp.int32)
            acc = acc + (plsc.load_gather(ubuf[p], [el, fv, lanes_u])
                         * plsc.load_gather(ibuf[p], [el, fv, lanes_i]))
        bv = plsc.load_gather(bbuf[p], [el, lanes_b])
        half = acc + bv + jnp.where(role == 0, w0vec, jnp.zeros((L,), jnp.float32))
        part_v[0, pl.ds(w * L, L)] = half

    fire(0, 0)

    def pairstep(hh, carry):
        w0i = 2 * hh
        fire(w0i + 1, 1)
        drain(0)
        compute(w0i, 0)

        @pl.when(hh < NWAVE // 2 - 1)
        def _():
            fire(w0i + 2, 0)

        drain(1)
        compute(w0i + 1, 1)
        return carry

    lax.fori_loop(0, NWAVE // 2, pairstep, 0)

    pltpu.sync_copy(part_v, sp_part.at[pl.ds(role, 1), pl.ds(q * EPP, EPP)])
    plsc.subcore_barrier()

    @pl.when(role == 0)
    def _():
        pltpu.sync_copy(sp_part.at[pl.ds(0, 1), pl.ds(q * EPP, EPP)], cb0)
        pltpu.sync_copy(sp_part.at[pl.ds(1, 1), pl.ds(q * EPP, EPP)], cb1)
        for g in range(NGR):
            sl = pl.ds(g * L, L)
            x = cb0[0, sl] + cb1[0, sl]
            out_v[sl] = 1.0 / (1.0 + jnp.exp(-x))
        pltpu.sync_copy(out_v, out_hbm.at[pl.ds(base, EPP)])


@jax.jit
def kernel(user, item, user_emb, item_emb, bias_table, w0):
    user1d = user.astype(jnp.int32)
    item1d = item.astype(jnp.int32)
    uembT = user_emb.T
    iembT = item_emb.T
    biasT = bias_table.T
    w0v = jnp.broadcast_to(w0.astype(jnp.float32), (L,))
    dummy = jnp.zeros((L, FH, 128), jnp.float32)

    fn = pl.kernel(
        _fm_body,
        out_type=jax.ShapeDtypeStruct((B,), jnp.float32),
        mesh=plsc.VectorSubcoreMesh(
            core_axis_name="c", subcore_axis_name="s",
            num_cores=NC, num_subcores=NS),
        scratch_types=[
            pltpu.VMEM((EPP,), jnp.int32),
            pltpu.VMEM((EPP,), jnp.int32),
            pltpu.VMEM((L, FH, 128), jnp.float32),
            pltpu.VMEM((L, FH, 128), jnp.float32),
            pltpu.VMEM((L, FH, 128), jnp.float32),
            pltpu.VMEM((L, FH, 128), jnp.float32),
            pltpu.VMEM((L, 128), jnp.float32),
            pltpu.VMEM((L, 128), jnp.float32),
            pltpu.VMEM((1, EPP), jnp.float32),
            pltpu.VMEM((1, EPP), jnp.float32),
            pltpu.VMEM((1, EPP), jnp.float32),
            pltpu.VMEM((L,), jnp.float32),
            pltpu.VMEM((EPP,), jnp.float32),
            pltpu.VMEM_SHARED((2, 8 * EPP), jnp.float32),
            pltpu.SemaphoreType.DMA,
            pltpu.SemaphoreType.DMA,
        ],
        compiler_params=pltpu.CompilerParams(needs_layout_passes=False),
    )
    return fn(user1d, item1d, uembT, iembT, biasT, w0v, dummy)


# speedup vs baseline: 1.0096x; 1.0096x over previous
import jax
import jax.numpy as jnp
from jax import lax
from jax.experimental import pallas as pl
from jax.experimental.pallas import tpu as pltpu
from jax.experimental.pallas import tpu_sc as plsc

B = 16384
F = 16
NC = 2
NS = 16
L = 16
NW = NC * NS
BPW = B // NW
NWAVE = BPW // L
NB = 1000064


def _fm_body(user_hbm, item_hbm, uembT_hbm, iembT_hbm, biasT_hbm, w0_hbm,
             out_hbm,
             idx_u, idx_i, ublk, iblk, bvals_u, bvals_i, w0_v, out_v,
             sp_bias, sem, sem_b):
    wid = lax.axis_index("s") * NC + lax.axis_index("c")
    base = wid * BPW
    sid = lax.axis_index("s")

    pltpu.sync_copy(user_hbm.at[pl.ds(base, BPW)], idx_u)
    pltpu.sync_copy(item_hbm.at[pl.ds(base, BPW)], idx_i)
    pltpu.sync_copy(w0_hbm, w0_v)

    @pl.when(sid == 0)
    def _():
        pltpu.sync_copy(biasT_hbm.at[0, pl.ds(0, NB)], sp_bias)
    plsc.subcore_barrier()

    bias_copies = []
    for j in range(BPW // 128):
        sl = pl.ds(j * 128, 128)
        bias_copies.append(pltpu.async_copy(sp_bias.at[idx_u.at[sl]], bvals_u.at[sl], sem_b))
        bias_copies.append(pltpu.async_copy(sp_bias.at[idx_i.at[sl]], bvals_i.at[sl], sem_b))
    for c in bias_copies:
        c.wait()

    w0vec = w0_v[...]
    mask127 = jnp.full((L,), 127, jnp.int32)

    def wave(w, carry):
        uvec = idx_u[pl.ds(w * L, L)]
        ivec = idx_i[pl.ds(w * L, L)]
        copies = []
        for j in range(L):
            ru = uvec[j]
            ri = ivec[j]
            rbu = pl.multiple_of(
                lax.shift_left(lax.shift_right_logical(ru, 7), 7), 128)
            rbi = pl.multiple_of(
                lax.shift_left(lax.shift_right_logical(ri, 7), 7), 128)
            copies.append(pltpu.async_copy(
                uembT_hbm.at[:, pl.ds(rbu, 128)], ublk.at[j], sem))
            copies.append(pltpu.async_copy(
                iembT_hbm.at[:, pl.ds(rbi, 128)], iblk.at[j], sem))
        for c in copies:
            c.wait()

        lanes_u = uvec & mask127
        lanes_i = ivec & mask127
        el = lax.iota(jnp.int32, L)
        acc = jnp.zeros((L,), jnp.float32)
        for f in range(F):
            fv = jnp.full((L,), f, jnp.int32)
            acc = acc + (plsc.load_gather(ublk, [el, fv, lanes_u])
                         * plsc.load_gather(iblk, [el, fv, lanes_i]))
        grow = w * L + el
        bu = plsc.load_gather(bvals_u, [grow])
        bi = plsc.load_gather(bvals_i, [grow])
        x = w0vec + bu + bi + acc
        out_v[pl.ds(w * L, L)] = 1.0 / (1.0 + jnp.exp(-x))
        return carry

    lax.fori_loop(0, NWAVE, wave, 0)
    pltpu.sync_copy(out_v, out_hbm.at[pl.ds(base, BPW)])


@jax.jit
def kernel(user, item, user_emb, item_emb, bias_table, w0):
    user1d = user.astype(jnp.int32)
    item1d = item.astype(jnp.int32)
    uembT = user_emb.T
    iembT = item_emb.T
    biasT = bias_table.T
    w0v = jnp.broadcast_to(w0.astype(jnp.float32), (L,))

    fn = pl.kernel(
        _fm_body,
        out_type=jax.ShapeDtypeStruct((B,), jnp.float32),
        mesh=plsc.VectorSubcoreMesh(
            core_axis_name="c", subcore_axis_name="s",
            num_cores=NC, num_subcores=NS),
        scratch_types=[
            pltpu.VMEM((BPW,), jnp.int32),
            pltpu.VMEM((BPW,), jnp.int32),
            pltpu.VMEM((L, F, 128), jnp.float32),
            pltpu.VMEM((L, F, 128), jnp.float32),
            pltpu.VMEM((BPW,), jnp.float32),
            pltpu.VMEM((BPW,), jnp.float32),
            pltpu.VMEM((L,), jnp.float32),
            pltpu.VMEM((BPW,), jnp.float32),
            pltpu.VMEM_SHARED((NB,), jnp.float32),
            pltpu.SemaphoreType.DMA,
            pltpu.SemaphoreType.DMA,
        ],
        compiler_params=pltpu.CompilerParams(needs_layout_passes=False),
    )
    return fn(user1d, item1d, uembT, iembT, biasT, w0v)
